# Initial kernel scaffold; baseline (speedup 1.0000x reference)
import jax
import jax.numpy as jnp
from jax.experimental import pallas as pl


def kernel(x, edge_index_1, edge_index_2, edge_index_3, edge_index_4, W1_n, W1_s, b1, W2_n, W2_s, b2, W3_n, W3_s, b3, W4_n, W4_s, b4, W_fc, b_fc):
    raise NotImplementedError("write your pallas kernel here")


# speedup vs baseline: 13.4971x; 13.4971x over previous
import functools

import jax
import jax.numpy as jnp
from jax import lax
from jax.experimental import pallas as pl
from jax.experimental.pallas import tpu as pltpu
from jax.experimental.pallas import tpu_sc as plsc

NC = 2
NS = 16
NW = NC * NS
CHUNK = 128


def _ceil_to(a: int, m: int) -> int:
    return -(-a // m) * m


@functools.lru_cache(maxsize=None)
def _make_seg_sum(N: int, F: int, n_chunks: int):
    N_pad = _ceil_to(N + 1, 256)
    rows_per = N_pad // NS
    mesh = plsc.VectorSubcoreMesh(core_axis_name="c", subcore_axis_name="s",
                                  num_cores=NC, num_subcores=NS)

    @functools.partial(
        pl.kernel,
        out_type=jax.ShapeDtypeStruct((NC, N_pad, F), jnp.float32),
        mesh=mesh,
        scratch_types=[
            pltpu.VMEM((n_chunks, CHUNK), jnp.int32),
            pltpu.VMEM((n_chunks, CHUNK), jnp.int32),
            pltpu.VMEM((CHUNK, F), jnp.float32),
            pltpu.VMEM_SHARED((N_pad, F), jnp.float32),
        ],
        compiler_params=pltpu.CompilerParams(use_tc_tiling_on_sc=False),
    )
    def seg_sum(x_hbm, src_hbm, dst_hbm, zeros_hbm, out_hbm,
                src_v, dst_v, rows_v, accum):
        c = lax.axis_index("c")
        s = lax.axis_index("s")
        wid = c * NS + s
        pltpu.sync_copy(zeros_hbm.at[pl.ds(s * rows_per, rows_per)],
                        accum.at[pl.ds(s * rows_per, rows_per)])
        pltpu.sync_copy(src_hbm.at[wid], src_v)
        pltpu.sync_copy(dst_hbm.at[wid], dst_v)
        plsc.subcore_barrier()

        def chunk_body(ci, carry):
            pltpu.sync_copy(x_hbm.at[src_v.at[ci]], rows_v)
            pltpu.sync_copy(rows_v, accum.at[dst_v.at[ci]], add=True)
            return carry

        lax.fori_loop(0, n_chunks, chunk_body, 0)
        plsc.subcore_barrier()
        pltpu.sync_copy(accum.at[pl.ds(s * rows_per, rows_per)],
                        out_hbm.at[c, pl.ds(s * rows_per, rows_per)])

    return seg_sum


def _seg_sum_level(x, edge_index, N, F):
    E = edge_index.shape[1]
    n_chunks = -(-E // (NW * CHUNK))
    E_pad = NW * n_chunks * CHUNK
    N_pad = _ceil_to(N + 1, 256)
    src = jnp.concatenate(
        [edge_index[0], jnp.zeros((E_pad - E,), jnp.int32)]).reshape(
            NW, n_chunks, CHUNK)
    dst = jnp.concatenate(
        [edge_index[1], jnp.full((E_pad - E,), N, jnp.int32)]).reshape(
            NW, n_chunks, CHUNK)
    zeros = jnp.zeros((N_pad, F), jnp.float32)
    return _make_seg_sum(N, F, n_chunks)(x, src, dst, zeros)


def _conv_body(agg_ref, x_ref, wn_ref, ws_ref, b_ref, o_ref, *, F, Wpad):
    a = agg_ref[0] + agg_ref[1]
    xb = x_ref[...]
    wn = wn_ref[...]
    ws = ws_ref[...]
    bb = b_ref[...]

    def half(i):
        ae = a[:, i * Wpad:i * Wpad + F]
        xe = xb[:, i * F:(i + 1) * F]
        if F == 1:
            h = ae * wn + xe * ws + bb
        else:
            h = (jnp.dot(ae, wn, preferred_element_type=jnp.float32)
                 + jnp.dot(xe, ws, preferred_element_type=jnp.float32) + bb)
        return h

    o_ref[...] = jnp.maximum(jnp.maximum(half(0), half(1)), 0.0)


def _conv_pool_tc(aggp, x_r, Wn, Ws, b, Nh, F, Wpad, Fout, R=512):
    grid = (-(-Nh // R),)
    return pl.pallas_call(
        functools.partial(_conv_body, F=F, Wpad=Wpad),
        grid=grid,
        in_specs=[
            pl.BlockSpec((NC, R, 2 * Wpad), lambda i: (0, i, 0)),
            pl.BlockSpec((R, 2 * F), lambda i: (i, 0)),
            pl.BlockSpec((F, Fout), lambda i: (0, 0)),
            pl.BlockSpec((F, Fout), lambda i: (0, 0)),
            pl.BlockSpec((1, Fout), lambda i: (0, 0)),
        ],
        out_specs=pl.BlockSpec((R, Fout), lambda i: (i, 0)),
        out_shape=jax.ShapeDtypeStruct((Nh, Fout), jnp.float32),
    )(aggp, x_r, Wn, Ws, b.reshape(1, Fout))


def _head_body(h_ref, w_ref, b_ref, o_ref, *, inv_n):
    s = jnp.sum(h_ref[...], axis=0, keepdims=True)
    o_ref[...] = (jnp.dot(s, w_ref[...], preferred_element_type=jnp.float32)
                  * inv_n + b_ref[...])


def _head_tc(h, W_fc, b_fc):
    n, _ = h.shape
    out = pl.pallas_call(
        functools.partial(_head_body, inv_n=1.0 / n),
        out_shape=jax.ShapeDtypeStruct((1, W_fc.shape[1]), jnp.float32),
    )(h, W_fc, b_fc.reshape(1, -1))
    return out[0]


F_SC_MIN = 8


def _level(x, edge_index, Wn, Ws, b):
    N, F = x.shape
    Fout = Wn.shape[1]
    Wpad = max(F, F_SC_MIN)
    xw = x if Wpad == F else jnp.pad(x, ((0, 0), (0, Wpad - F)))
    agg = _seg_sum_level(xw, edge_index, N, Wpad)
    aggp = agg.reshape(NC, agg.shape[1] // 2, 2 * Wpad)
    x_r = x.reshape(N // 2, 2 * F)
    return _conv_pool_tc(aggp, x_r, Wn, Ws, b, N // 2, F, Wpad, Fout)


def kernel(x, edge_index_1, edge_index_2, edge_index_3, edge_index_4,
           W1_n, W1_s, b1, W2_n, W2_s, b2, W3_n, W3_s, b3, W4_n, W4_s, b4,
           W_fc, b_fc):
    h = _level(x, edge_index_1, W1_n, W1_s, b1)
    h = _level(h, edge_index_2, W2_n, W2_s, b2)
    h = _level(h, edge_index_3, W3_n, W3_s, b3)
    h = _level(h, edge_index_4, W4_n, W4_s, b4)
    return _head_tc(h, W_fc, b_fc)
